# baseline (speedup 1.0000x reference)
import jax
import jax.numpy as jnp
from jax.experimental import pallas as pl


def _fused_mlp_kernel(x_ref, w1g_ref, w1u_ref, b1g_ref, b1u_ref,
                      w2_ref, b2_ref, out_ref):
    x = x_ref[0, 0]
    w1g = w1g_ref[0]
    w1u = w1u_ref[0]
    g = jnp.dot(x, w1g, preferred_element_type=jnp.float32) + b1g_ref[0]
    u = jnp.dot(x, w1u, preferred_element_type=jnp.float32) + b1u_ref[0]
    act = (g * jax.nn.sigmoid(g)) * u
    y = jnp.dot(act, w2_ref[0], preferred_element_type=jnp.float32) + b2_ref[0]
    out_ref[0, 0] = y


def kernel(dispatched, sparsity_remap, w1, w1_bias, w2, w2_bias):
    A, B, E, M, K = dispatched.shape
    N2 = w1.shape[2]
    inter = N2 // 2
    H = w2.shape[2]

    w1g = w1[:, :, 0::2]
    w1u = w1[:, :, 1::2]
    b1g = w1_bias[:, 0::2].reshape(E, 1, inter)
    b1u = w1_bias[:, 1::2].reshape(E, 1, inter)
    b2 = w2_bias.reshape(E, 1, H)

    x = dispatched.reshape(A, E, M, K)

    TM = min(512, M)
    mt = M // TM
    grid = (E, A, mt)

    out = pl.pallas_call(
        _fused_mlp_kernel,
        grid=grid,
        in_specs=[
            pl.BlockSpec((1, 1, TM, K), lambda e, a, m: (a, e, m, 0)),
            pl.BlockSpec((1, K, inter), lambda e, a, m: (e, 0, 0)),
            pl.BlockSpec((1, K, inter), lambda e, a, m: (e, 0, 0)),
            pl.BlockSpec((1, 1, inter), lambda e, a, m: (e, 0, 0)),
            pl.BlockSpec((1, 1, inter), lambda e, a, m: (e, 0, 0)),
            pl.BlockSpec((1, inter, H), lambda e, a, m: (e, 0, 0)),
            pl.BlockSpec((1, 1, H), lambda e, a, m: (e, 0, 0)),
        ],
        out_specs=pl.BlockSpec((1, 1, TM, H),
                               lambda e, a, m: (e, 0, a * (M // TM) + m, 0)),
        out_shape=jax.ShapeDtypeStruct((E, 1, A * M, H), jnp.float32),
    )(x, w1g, w1u, b1g, b1u, w2, b2)

    return out


# speedup vs baseline: 1.7525x; 1.7525x over previous
import jax
import jax.numpy as jnp
from jax.experimental import pallas as pl


def _fused_mlp_kernel(x_ref, w1g_ref, w1u_ref, b1g_ref, b1u_ref,
                      w2_ref, b2_ref, out_ref):
    x = x_ref[0, 0]
    w1g = w1g_ref[0]
    w1u = w1u_ref[0]
    g = jnp.dot(x, w1g, preferred_element_type=jnp.float32) + b1g_ref[0]
    u = jnp.dot(x, w1u, preferred_element_type=jnp.float32) + b1u_ref[0]
    act = (g * jax.nn.sigmoid(g)) * u
    y = jnp.dot(act.astype(jnp.bfloat16), w2_ref[0],
                preferred_element_type=jnp.float32) + b2_ref[0]
    out_ref[0, 0] = y


def kernel(dispatched, sparsity_remap, w1, w1_bias, w2, w2_bias):
    A, B, E, M, K = dispatched.shape
    N2 = w1.shape[2]
    inter = N2 // 2
    H = w2.shape[2]

    w1g = w1[:, :, 0::2].astype(jnp.bfloat16)
    w1u = w1[:, :, 1::2].astype(jnp.bfloat16)
    b1g = w1_bias[:, 0::2].reshape(E, 1, inter)
    b1u = w1_bias[:, 1::2].reshape(E, 1, inter)
    b2 = w2_bias.reshape(E, 1, H)

    x = dispatched.reshape(A, E, M, K).astype(jnp.bfloat16)
    w2 = w2.astype(jnp.bfloat16)

    TM = min(512, M)
    mt = M // TM
    grid = (E, A, mt)

    out = pl.pallas_call(
        _fused_mlp_kernel,
        grid=grid,
        in_specs=[
            pl.BlockSpec((1, 1, TM, K), lambda e, a, m: (a, e, m, 0)),
            pl.BlockSpec((1, K, inter), lambda e, a, m: (e, 0, 0)),
            pl.BlockSpec((1, K, inter), lambda e, a, m: (e, 0, 0)),
            pl.BlockSpec((1, 1, inter), lambda e, a, m: (e, 0, 0)),
            pl.BlockSpec((1, 1, inter), lambda e, a, m: (e, 0, 0)),
            pl.BlockSpec((1, inter, H), lambda e, a, m: (e, 0, 0)),
            pl.BlockSpec((1, 1, H), lambda e, a, m: (e, 0, 0)),
        ],
        out_specs=pl.BlockSpec((1, 1, TM, H),
                               lambda e, a, m: (e, 0, a * (M // TM) + m, 0)),
        out_shape=jax.ShapeDtypeStruct((E, 1, A * M, H), jnp.float32),
    )(x, w1g, w1u, b1g, b1u, w2, b2)

    return out


# speedup vs baseline: 5.6462x; 3.2218x over previous
import jax
import jax.numpy as jnp
from jax.experimental import pallas as pl


def _fused_mlp_kernel(x_ref, w1_ref, b1_ref, w2p_ref, b2_ref, out_ref):
    x = x_ref[0, 0]
    h = jnp.dot(x, w1_ref[0], preferred_element_type=jnp.float32) + b1_ref[0]
    hr = jnp.roll(h, shift=-1, axis=1)
    s = (h * jax.nn.sigmoid(h)) * hr
    y = jnp.dot(s.astype(jnp.bfloat16), w2p_ref[0],
                preferred_element_type=jnp.float32) + b2_ref[0]
    out_ref[0, 0] = y


def kernel(dispatched, sparsity_remap, w1, w1_bias, w2, w2_bias):
    A, B, E, M, K = dispatched.shape
    N2 = w1.shape[2]
    inter = N2 // 2
    H = w2.shape[2]

    b1 = w1_bias.reshape(E, 1, N2)
    b2 = w2_bias.reshape(E, 1, H)

    x = dispatched.reshape(A * B, E, M, K).astype(jnp.bfloat16)
    w1 = w1.astype(jnp.bfloat16)
    w2p = jnp.stack(
        [w2.astype(jnp.bfloat16), jnp.zeros((E, inter, H), jnp.bfloat16)],
        axis=2,
    ).reshape(E, N2, H)

    TM = min(512, M)
    mt = M // TM
    grid = (E, A * B, mt)

    out = pl.pallas_call(
        _fused_mlp_kernel,
        grid=grid,
        in_specs=[
            pl.BlockSpec((1, 1, TM, K), lambda e, a, m: (a, e, m, 0)),
            pl.BlockSpec((1, K, N2), lambda e, a, m: (e, 0, 0)),
            pl.BlockSpec((1, 1, N2), lambda e, a, m: (e, 0, 0)),
            pl.BlockSpec((1, N2, H), lambda e, a, m: (e, 0, 0)),
            pl.BlockSpec((1, 1, H), lambda e, a, m: (e, 0, 0)),
        ],
        out_specs=pl.BlockSpec((1, 1, TM, H),
                               lambda e, a, m: (e, 0, a * mt + m, 0)),
        out_shape=jax.ShapeDtypeStruct((E, 1, A * B * M, H), jnp.float32),
    )(x, w1, b1, w2p, b2)

    return out


# speedup vs baseline: 5.9778x; 1.0587x over previous
import jax
import jax.numpy as jnp
from jax.experimental import pallas as pl
from jax.experimental.pallas import tpu as pltpu


def _fused_mlp_kernel(x_ref, w1_ref, b1_ref, w2p_ref, b2_ref, out_ref, w1b_s):
    a = pl.program_id(1)
    m = pl.program_id(2)

    @pl.when(jnp.logical_and(a == 0, m == 0))
    def _cast_w1():
        w1b_s[...] = w1_ref[0].astype(jnp.bfloat16)

    x = x_ref[0, 0].astype(jnp.bfloat16)
    h = jnp.dot(x, w1b_s[...], preferred_element_type=jnp.float32) + b1_ref[0]
    hr = jnp.roll(h, shift=-1, axis=1)
    s = (h * jax.nn.sigmoid(h)) * hr
    y = jnp.dot(s.astype(jnp.bfloat16), w2p_ref[0],
                preferred_element_type=jnp.float32) + b2_ref[0]
    out_ref[0, 0] = y


def kernel(dispatched, sparsity_remap, w1, w1_bias, w2, w2_bias):
    A, B, E, M, K = dispatched.shape
    N2 = w1.shape[2]
    inter = N2 // 2
    H = w2.shape[2]

    b1 = w1_bias.reshape(E, 1, N2)
    b2 = w2_bias.reshape(E, 1, H)

    x = dispatched.reshape(A * B, E, M, K)
    w2p = jnp.stack(
        [w2.astype(jnp.bfloat16), jnp.zeros((E, inter, H), jnp.bfloat16)],
        axis=2,
    ).reshape(E, N2, H)

    TM = min(512, M)
    mt = M // TM
    grid = (E, A * B, mt)

    out = pl.pallas_call(
        _fused_mlp_kernel,
        grid=grid,
        in_specs=[
            pl.BlockSpec((1, 1, TM, K), lambda e, a, m: (a, e, m, 0)),
            pl.BlockSpec((1, K, N2), lambda e, a, m: (e, 0, 0)),
            pl.BlockSpec((1, 1, N2), lambda e, a, m: (e, 0, 0)),
            pl.BlockSpec((1, N2, H), lambda e, a, m: (e, 0, 0)),
            pl.BlockSpec((1, 1, H), lambda e, a, m: (e, 0, 0)),
        ],
        out_specs=pl.BlockSpec((1, 1, TM, H),
                               lambda e, a, m: (e, 0, a * mt + m, 0)),
        out_shape=jax.ShapeDtypeStruct((E, 1, A * B * M, H), jnp.float32),
        scratch_shapes=[pltpu.VMEM((K, N2), jnp.bfloat16)],
    )(x, w1, b1, w2p, b2)

    return out


# speedup vs baseline: 6.1928x; 1.0360x over previous
import jax
import jax.numpy as jnp
from jax.experimental import pallas as pl
from jax.experimental.pallas import tpu as pltpu


def _fused_mlp_kernel(x_ref, w1_ref, b1_ref, w2p_ref, b2_ref, out_ref, w1b_s):
    a = pl.program_id(1)
    m = pl.program_id(2)

    @pl.when(jnp.logical_and(a == 0, m == 0))
    def _cast_w1():
        w1b_s[...] = w1_ref[0].astype(jnp.bfloat16)

    x = x_ref[0, 0].astype(jnp.bfloat16)
    h = jnp.dot(x, w1b_s[...], preferred_element_type=jnp.float32) + b1_ref[0]
    hr = jnp.roll(h, shift=-1, axis=1)
    s = (h * jax.nn.sigmoid(h)) * hr
    y = jnp.dot(s.astype(jnp.bfloat16), w2p_ref[0],
                preferred_element_type=jnp.float32) + b2_ref[0]
    out_ref[0, 0] = y


def kernel(dispatched, sparsity_remap, w1, w1_bias, w2, w2_bias):
    A, B, E, M, K = dispatched.shape
    N2 = w1.shape[2]
    inter = N2 // 2
    H = w2.shape[2]

    b1 = w1_bias.reshape(E, 1, N2)
    b2 = w2_bias.reshape(E, 1, H)

    x = dispatched.reshape(A * B, E, M, K)
    w2p = jnp.stack(
        [w2.astype(jnp.bfloat16), jnp.zeros((E, inter, H), jnp.bfloat16)],
        axis=2,
    ).reshape(E, N2, H)

    TM = min(1024, M)
    mt = M // TM
    grid = (E, A * B, mt)

    out = pl.pallas_call(
        _fused_mlp_kernel,
        grid=grid,
        in_specs=[
            pl.BlockSpec((1, 1, TM, K), lambda e, a, m: (a, e, m, 0)),
            pl.BlockSpec((1, K, N2), lambda e, a, m: (e, 0, 0)),
            pl.BlockSpec((1, 1, N2), lambda e, a, m: (e, 0, 0)),
            pl.BlockSpec((1, N2, H), lambda e, a, m: (e, 0, 0)),
            pl.BlockSpec((1, 1, H), lambda e, a, m: (e, 0, 0)),
        ],
        out_specs=pl.BlockSpec((1, 1, TM, H),
                               lambda e, a, m: (e, 0, a * mt + m, 0)),
        out_shape=jax.ShapeDtypeStruct((E, 1, A * B * M, H), jnp.float32),
        scratch_shapes=[pltpu.VMEM((K, N2), jnp.bfloat16)],
    )(x, w1, b1, w2p, b2)

    return out


# speedup vs baseline: 14.6273x; 2.3620x over previous
import jax
import jax.numpy as jnp
from jax.experimental import pallas as pl
from jax.experimental.pallas import tpu as pltpu


def _fused_mlp_kernel(x_ref, w1_ref, b1_ref, w2_ref, b2_ref, out_ref, w2p_s):
    a = pl.program_id(1)
    m = pl.program_id(2)

    @pl.when(jnp.logical_and(a == 0, m == 0))
    def _prep_weights():
        bits = pltpu.bitcast(w2_ref[0], jnp.int32)
        b16 = (bits + 0x7FFF + ((bits >> 16) & 1)) >> 16
        w2p_s[...] = pltpu.bitcast((b16 & 0xFFFF).astype(jnp.int32),
                                   jnp.bfloat16)

    x = x_ref[0, 0].astype(jnp.bfloat16)
    h = jnp.dot(x, w1_ref[0].astype(jnp.bfloat16),
                preferred_element_type=jnp.float32) + b1_ref[0]
    hr = jnp.roll(h, shift=-1, axis=1)
    s = (h * jax.nn.sigmoid(h)) * hr
    y = jnp.dot(s.astype(jnp.bfloat16), w2p_s[...],
                preferred_element_type=jnp.float32) + b2_ref[0]
    out_ref[0, 0] = y


def kernel(dispatched, sparsity_remap, w1, w1_bias, w2, w2_bias):
    A, B, E, M, K = dispatched.shape
    N2 = w1.shape[2]
    inter = N2 // 2
    H = w2.shape[2]

    b1 = w1_bias.reshape(E, 1, N2)
    b2 = w2_bias.reshape(E, 1, H)

    x = dispatched.reshape(A * B, E, M, K)

    TM = min(1024, M)
    mt = M // TM
    grid = (E, A * B, mt)

    out = pl.pallas_call(
        _fused_mlp_kernel,
        grid=grid,
        in_specs=[
            pl.BlockSpec((1, 1, TM, K), lambda e, a, m: (a, e, m, 0)),
            pl.BlockSpec((1, K, N2), lambda e, a, m: (e, 0, 0)),
            pl.BlockSpec((1, 1, N2), lambda e, a, m: (e, 0, 0)),
            pl.BlockSpec((1, inter, H), lambda e, a, m: (e, 0, 0)),
            pl.BlockSpec((1, 1, H), lambda e, a, m: (e, 0, 0)),
        ],
        out_specs=pl.BlockSpec((1, 1, TM, H),
                               lambda e, a, m: (e, 0, a * mt + m, 0)),
        out_shape=jax.ShapeDtypeStruct((E, 1, A * B * M, H), jnp.float32),
        scratch_shapes=[pltpu.VMEM((N2, H), jnp.bfloat16)],
    )(x, w1, b1, w2, b2)

    return out
